# Initial kernel scaffold; baseline (speedup 1.0000x reference)
import jax
import jax.numpy as jnp
from jax.experimental import pallas as pl


def kernel(x, edge_index, W1, a_src1, a_dst1, b1, W2, a_src2, a_dst2, b2, W3, a_src3, a_dst3, b3, W4, a_src4, a_dst4, b4, Wc1, bc1, Wc2, bc2):
    raise NotImplementedError("write your pallas kernel here")


# speedup vs baseline: 26.2625x; 26.2625x over previous
import functools

import jax
import jax.numpy as jnp
from jax import lax
from jax.experimental import pallas as pl
from jax.experimental.pallas import tpu as pltpu
from jax.experimental.pallas import tpu_sc as plsc

N = 10000
E = 320000
EAUG = E + N
D_IN, D_HID, D_LAT, N_CLS = 128, 128, 64, 10

NP = 10240
BM = 256
NBLK = NP // BM

NC, NS = 2, 16
NW = NC * NS
K = 128
NCHUNK = -(-EAUG // (NW * K))
TPW = NCHUNK * K
ETOT = NW * TPW
ROWS_PER_TILE = NP // NS

_f32 = jnp.float32


def _make_sc_edge(C):
  mesh = plsc.VectorSubcoreMesh(core_axis_name="c", subcore_axis_name="s")

  def body(h_hbm, asad_hbm, src_hbm, dst_hbm,
           acc_out, den_out,
           asad_v, src_v, dst_v, ex_v, rows_v, den_v, acc_sh, sem):
    c = lax.axis_index("c")
    s = lax.axis_index("s")
    wid = s * NC + c

    pltpu.sync_copy(asad_hbm, asad_v)

    zeros16 = jnp.zeros((16,), _f32)
    def _zden(i, carry):
      den_v[pl.ds(i * 16, 16)] = zeros16
      return carry
    lax.fori_loop(0, NP // 16, _zden, 0, unroll=8)

    def _zrow(i, carry):
      rows_v[i // (C // 16), pl.ds((i % (C // 16)) * 16, 16)] = zeros16
      return carry
    lax.fori_loop(0, K * C // 16, _zrow, 0, unroll=8)
    rbase = s * ROWS_PER_TILE
    def _zacc(i, carry):
      pltpu.sync_copy(rows_v, acc_sh.at[pl.ds(rbase + i * K, K)])
      return carry
    lax.fori_loop(0, ROWS_PER_TILE // K, _zacc, 0)
    rem = ROWS_PER_TILE % K
    if rem:
      pltpu.sync_copy(rows_v.at[pl.ds(0, rem)],
                      acc_sh.at[pl.ds(rbase + (ROWS_PER_TILE // K) * K, rem)])
    plsc.subcore_barrier()

    ebase0 = wid * TPW

    def _chunk(j, carry):
      ebase = ebase0 + j * K
      pltpu.sync_copy(src_hbm.at[pl.ds(ebase, K)], src_v)
      pltpu.sync_copy(dst_hbm.at[pl.ds(ebase, K)], dst_v)
      gather = pltpu.make_async_copy(h_hbm.at[src_v], rows_v, sem)
      gather.start()

      def _alpha(i, carry2):
        sv = src_v[pl.ds(i * 16, 16)]
        dv = dst_v[pl.ds(i * 16, 16)]
        ag = plsc.load_gather(asad_v, [sv * 2])
        dg = plsc.load_gather(asad_v, [dv * 2 + 1])
        al = ag + dg
        al = jnp.maximum(al, al * 0.2)
        ex = jnp.exp(al)
        ex_v[pl.ds(i * 16, 16)] = ex
        plsc.addupdate_scatter(den_v, [dv], ex)
        return carry2
      lax.fori_loop(0, K // 16, _alpha, 0)

      gather.wait()

      def _scale(i, carry2):
        ex16 = ex_v[pl.ds(i * 16, 16)]
        for j in range(16):
          e = ex16[j]
          r = i * 16 + j
          for cc in range(C // 16):
            sl = pl.ds(cc * 16, 16)
            rows_v[r, sl] = rows_v[r, sl] * e
        return carry2
      lax.fori_loop(0, K // 16, _scale, 0)

      pltpu.sync_copy(rows_v, acc_sh.at[dst_v], add=True)
      return carry
    lax.fori_loop(0, NCHUNK, _chunk, 0)

    plsc.subcore_barrier()
    pltpu.sync_copy(acc_sh.at[pl.ds(rbase, ROWS_PER_TILE)],
                    acc_out.at[c, pl.ds(rbase, ROWS_PER_TILE)])
    pltpu.sync_copy(den_v, den_out.at[wid])

  return pl.kernel(
      body,
      out_type=(jax.ShapeDtypeStruct((NC, NP, C), _f32),
                jax.ShapeDtypeStruct((NW, NP), _f32)),
      mesh=mesh,
      scratch_types=[
          pltpu.VMEM((2 * NP,), _f32),
          pltpu.VMEM((K,), jnp.int32),
          pltpu.VMEM((K,), jnp.int32),
          pltpu.VMEM((K,), _f32),
          pltpu.VMEM((K, C), _f32),
          pltpu.VMEM((NP,), _f32),
          pltpu.VMEM_SHARED((NP, C), _f32),
          pltpu.SemaphoreType.DMA,
      ],
      compiler_params=pltpu.CompilerParams(needs_layout_passes=False),
      name=f"sc_gat_edge_c{C}",
  )


def _tc_first(x, W, asrc, adst):
  Cin, C = W.shape

  def body(x_ref, w_ref, asrc_ref, adst_ref, h_ref, asad_ref):
    h = jnp.dot(x_ref[...], w_ref[...], preferred_element_type=_f32,
                precision=lax.Precision.HIGHEST)
    h_ref[...] = h
    a_s = jnp.sum(h * asrc_ref[...], axis=1)
    a_d = jnp.sum(h * adst_ref[...], axis=1)
    asad_ref[...] = jnp.stack([a_s, a_d], axis=1)

  return pl.pallas_call(
      body,
      grid=(NBLK,),
      in_specs=[
          pl.BlockSpec((BM, Cin), lambda i: (i, 0)),
          pl.BlockSpec((Cin, C), lambda i: (0, 0)),
          pl.BlockSpec((1, C), lambda i: (0, 0)),
          pl.BlockSpec((1, C), lambda i: (0, 0)),
      ],
      out_specs=[
          pl.BlockSpec((BM, C), lambda i: (i, 0)),
          pl.BlockSpec((BM, 2), lambda i: (i, 0)),
      ],
      out_shape=[
          jax.ShapeDtypeStruct((NP, C), _f32),
          jax.ShapeDtypeStruct((NP, 2), _f32),
      ],
      name="tc_first",
  )(x, W, asrc.reshape(1, C), adst.reshape(1, C))


def _tc_combine_stage(acc, den, b, W, asrc, adst, *, relu, want_x,
                      want_latsum=False):
  Cin, C = W.shape

  def body(acc_ref, den_ref, b_ref, w_ref, asrc_ref, adst_ref, *outs):
    dsum = jnp.sum(den_ref[...], axis=0)
    x = (acc_ref[0] + acc_ref[1]) / (dsum[:, None] + 1e-16) + b_ref[...]
    if relu:
      x = jnp.maximum(x, 0.0)
    h = jnp.dot(x, w_ref[...], preferred_element_type=_f32,
                precision=lax.Precision.HIGHEST)
    a_s = jnp.sum(h * asrc_ref[...], axis=1)
    a_d = jnp.sum(h * adst_ref[...], axis=1)
    o = 0
    outs[o][...] = h; o += 1
    outs[o][...] = jnp.stack([a_s, a_d], axis=1); o += 1
    if want_x:
      outs[o][...] = x; o += 1
    if want_latsum:
      rid = pl.program_id(0) * BM + lax.broadcasted_iota(jnp.int32, (BM, 1), 0)
      xm = jnp.where(rid < N, x, 0.0)
      bsum = jnp.dot(jnp.ones((8, BM), _f32) / 8.0, xm,
                     preferred_element_type=_f32,
                     precision=lax.Precision.HIGHEST)
      outs[o][...] = bsum; o += 1

  out_specs = [
      pl.BlockSpec((BM, C), lambda i: (i, 0)),
      pl.BlockSpec((BM, 2), lambda i: (i, 0)),
  ]
  out_shape = [
      jax.ShapeDtypeStruct((NP, C), _f32),
      jax.ShapeDtypeStruct((NP, 2), _f32),
  ]
  if want_x:
    out_specs.append(pl.BlockSpec((BM, Cin), lambda i: (i, 0)))
    out_shape.append(jax.ShapeDtypeStruct((NP, Cin), _f32))
  if want_latsum:
    out_specs.append(pl.BlockSpec((8, Cin), lambda i: (i, 0)))
    out_shape.append(jax.ShapeDtypeStruct((NBLK * 8, Cin), _f32))

  return pl.pallas_call(
      body,
      grid=(NBLK,),
      in_specs=[
          pl.BlockSpec((NC, BM, Cin), lambda i: (0, i, 0)),
          pl.BlockSpec((NW, BM), lambda i: (0, i)),
          pl.BlockSpec((1, Cin), lambda i: (0, 0)),
          pl.BlockSpec((Cin, C), lambda i: (0, 0)),
          pl.BlockSpec((1, C), lambda i: (0, 0)),
          pl.BlockSpec((1, C), lambda i: (0, 0)),
      ],
      out_specs=out_specs,
      out_shape=out_shape,
      name="tc_combine_mm",
  )(acc, den, b.reshape(1, Cin), W, asrc.reshape(1, C), adst.reshape(1, C))


def _tc_final_combine(acc, den, b):
  C = acc.shape[-1]

  def body(acc_ref, den_ref, b_ref, o_ref):
    dsum = jnp.sum(den_ref[...], axis=0)
    o_ref[...] = (acc_ref[0] + acc_ref[1]) / (dsum[:, None] + 1e-16) + b_ref[...]

  return pl.pallas_call(
      body,
      grid=(NBLK,),
      in_specs=[
          pl.BlockSpec((NC, BM, C), lambda i: (0, i, 0)),
          pl.BlockSpec((NW, BM), lambda i: (0, i)),
          pl.BlockSpec((1, C), lambda i: (0, 0)),
      ],
      out_specs=pl.BlockSpec((BM, C), lambda i: (i, 0)),
      out_shape=jax.ShapeDtypeStruct((NP, C), _f32),
      name="tc_final_combine",
  )(acc, den, b.reshape(1, C))


def _tc_classifier(latsum, Wc1, bc1, Wc2, bc2):
  def body(ls_ref, w1_ref, b1_ref, w2_ref, b2_ref, o_ref):
    pooled = jnp.sum(ls_ref[...], axis=0, keepdims=True) / float(N)
    z = jnp.maximum(
        jnp.dot(pooled, w1_ref[...], preferred_element_type=_f32,
                precision=lax.Precision.HIGHEST) + b1_ref[...], 0.0)
    o_ref[...] = jnp.dot(z, w2_ref[...], preferred_element_type=_f32,
                         precision=lax.Precision.HIGHEST) + b2_ref[...]

  return pl.pallas_call(
      body,
      out_shape=jax.ShapeDtypeStruct((1, N_CLS), _f32),
      name="tc_classifier",
  )(latsum, Wc1, bc1.reshape(1, -1), Wc2, bc2.reshape(1, -1))


_sc_edge_128 = _make_sc_edge(128)


@jax.jit
def kernel(x, edge_index, W1, a_src1, a_dst1, b1, W2, a_src2, a_dst2, b2,
           W3, a_src3, a_dst3, b3, W4, a_src4, a_dst4, b4, Wc1, bc1, Wc2, bc2):
  W2 = jnp.pad(W2, ((0, 0), (0, 64)))
  a_src2 = jnp.pad(a_src2, (0, 64))
  a_dst2 = jnp.pad(a_dst2, (0, 64))
  b2 = jnp.pad(b2, (0, 64))
  W3 = jnp.pad(W3, ((0, 64), (0, 0)))
  Wc1 = jnp.pad(Wc1, ((0, 64), (0, 0)))
  x_pad = jnp.zeros((NP, D_IN), _f32).at[:N].set(x)
  loops = jnp.arange(N, dtype=jnp.int32)
  src = jnp.full((ETOT,), N, jnp.int32).at[:EAUG].set(
      jnp.concatenate([edge_index[0], loops]))
  dst = jnp.full((ETOT,), N, jnp.int32).at[:EAUG].set(
      jnp.concatenate([edge_index[1], loops]))

  h1, asad1 = _tc_first(x_pad, W1, a_src1, a_dst1)
  acc1, den1 = _sc_edge_128(h1, asad1.reshape(-1), src, dst)
  h2, asad2 = _tc_combine_stage(acc1, den1, b1, W2, a_src2, a_dst2,
                                relu=True, want_x=False)
  acc2, den2 = _sc_edge_128(h2, asad2.reshape(-1), src, dst)
  h3, asad3, latent_full, latsum = _tc_combine_stage(
      acc2, den2, b2, W3, a_src3, a_dst3, relu=False, want_x=True,
      want_latsum=True)
  acc3, den3 = _sc_edge_128(h3, asad3.reshape(-1), src, dst)
  h4, asad4 = _tc_combine_stage(acc3, den3, b3, W4, a_src4, a_dst4,
                                relu=True, want_x=False)
  acc4, den4 = _sc_edge_128(h4, asad4.reshape(-1), src, dst)
  recon_full = _tc_final_combine(acc4, den4, b4)

  noise_pred = _tc_classifier(latsum, Wc1, bc1, Wc2, bc2)

  return recon_full[:N], latent_full[:N, :D_LAT], noise_pred


# speedup vs baseline: 32.5075x; 1.2378x over previous
import functools

import jax
import jax.numpy as jnp
from jax import lax
from jax.experimental import pallas as pl
from jax.experimental.pallas import tpu as pltpu
from jax.experimental.pallas import tpu_sc as plsc

N = 10000
E = 320000
EAUG = E + N
D_IN, D_HID, D_LAT, N_CLS = 128, 128, 64, 10

NP = 10240
BM = 256
NBLK = NP // BM

NC, NS = 2, 16
NW = NC * NS
K = 96
NCHUNK = 2 * (-(-EAUG // (NW * K * 2)))
TPW = NCHUNK * K
ETOT = NW * TPW
ROWS_PER_TILE = NP // NS

_f32 = jnp.float32


def _make_sc_edge(C):
  mesh = plsc.VectorSubcoreMesh(core_axis_name="c", subcore_axis_name="s")

  def body(h_hbm, asad_hbm, src_hbm, dst_hbm,
           acc_out, den_out,
           asad_v, src0_v, dst0_v, src1_v, dst1_v, ex_v,
           rows0_v, rows1_v, acc_sh, den_sh, sem0, sem1):
    c = lax.axis_index("c")
    s = lax.axis_index("s")
    wid = s * NC + c

    pltpu.sync_copy(asad_hbm, asad_v)

    zeros16 = jnp.zeros((16,), _f32)
    def _zrow(i, carry):
      rows0_v[i // (C // 16), pl.ds((i % (C // 16)) * 16, 16)] = zeros16
      return carry
    lax.fori_loop(0, K * C // 16, _zrow, 0, unroll=8)
    rbase = s * ROWS_PER_TILE
    def _zacc(i, carry):
      pltpu.sync_copy(rows0_v, acc_sh.at[pl.ds(rbase + i * K, K)])
      return carry
    lax.fori_loop(0, ROWS_PER_TILE // K, _zacc, 0)
    rem = ROWS_PER_TILE % K
    if rem:
      pltpu.sync_copy(rows0_v.at[pl.ds(0, rem)],
                      acc_sh.at[pl.ds(rbase + (ROWS_PER_TILE // K) * K, rem)])
    flat0 = rows0_v.at[0]
    assert ROWS_PER_TILE % C == 0
    def _zden(i, carry):
      pltpu.sync_copy(flat0, den_sh.at[pl.ds(rbase + i * C, C)])
      return carry
    lax.fori_loop(0, ROWS_PER_TILE // C, _zden, 0)
    plsc.subcore_barrier()

    ebase0 = wid * TPW

    def _stage_idx(j, sv, dv):
      ebase = ebase0 + j * K
      pltpu.sync_copy(src_hbm.at[pl.ds(ebase, K)], sv)
      pltpu.sync_copy(dst_hbm.at[pl.ds(ebase, K)], dv)

    def _alpha(sv_ref, dv_ref):
      def step(i, carry):
        sv = sv_ref[pl.ds(i * 16, 16)]
        dv = dv_ref[pl.ds(i * 16, 16)]
        ag = plsc.load_gather(asad_v, [sv * 2])
        dg = plsc.load_gather(asad_v, [dv * 2 + 1])
        al = ag + dg
        al = jnp.maximum(al, al * 0.2)
        ex = jnp.exp(al)
        ex_v[pl.ds(i * 16, 16)] = ex
        return carry
      lax.fori_loop(0, K // 16, step, 0)

    def _scale(rows_ref):
      def step(i, carry):
        ex16 = ex_v[pl.ds(i * 16, 16)]
        for jj in range(16):
          e = ex16[jj]
          r = i * 16 + jj
          for cc in range(C // 16):
            sl = pl.ds(cc * 16, 16)
            rows_ref[r, sl] = rows_ref[r, sl] * e
        return carry
      lax.fori_loop(0, K // 16, step, 0)

    def _process(rows_ref, sv_ref, dv_ref, sem):
      _alpha(sv_ref, dv_ref)
      pltpu.sync_copy(ex_v, den_sh.at[dv_ref], add=True)
      pltpu.make_async_copy(h_hbm.at[sv_ref], rows_ref, sem).wait()
      _scale(rows_ref)
      pltpu.sync_copy(rows_ref, acc_sh.at[dv_ref], add=True)

    _stage_idx(0, src0_v, dst0_v)
    pltpu.make_async_copy(h_hbm.at[src0_v], rows0_v, sem0).start()
    _stage_idx(1, src1_v, dst1_v)
    pltpu.make_async_copy(h_hbm.at[src1_v], rows1_v, sem1).start()

    def _super(t, carry):
      j2 = 2 * t + 2
      _process(rows0_v, src0_v, dst0_v, sem0)
      @pl.when(j2 < NCHUNK)
      def _():
        _stage_idx(j2, src0_v, dst0_v)
        pltpu.make_async_copy(h_hbm.at[src0_v], rows0_v, sem0).start()
      _process(rows1_v, src1_v, dst1_v, sem1)
      @pl.when(j2 + 1 < NCHUNK)
      def _():
        _stage_idx(j2 + 1, src1_v, dst1_v)
        pltpu.make_async_copy(h_hbm.at[src1_v], rows1_v, sem1).start()
      return carry
    lax.fori_loop(0, NCHUNK // 2, _super, 0)

    plsc.subcore_barrier()
    pltpu.sync_copy(acc_sh.at[pl.ds(rbase, ROWS_PER_TILE)],
                    acc_out.at[c, pl.ds(rbase, ROWS_PER_TILE)])
    pltpu.sync_copy(den_sh.at[pl.ds(rbase, ROWS_PER_TILE)],
                    den_out.at[c, pl.ds(rbase, ROWS_PER_TILE)])

  return pl.kernel(
      body,
      out_type=(jax.ShapeDtypeStruct((NC, NP, C), _f32),
                jax.ShapeDtypeStruct((NC, NP), _f32)),
      mesh=mesh,
      scratch_types=[
          pltpu.VMEM((2 * NP,), _f32),
          pltpu.VMEM((K,), jnp.int32),
          pltpu.VMEM((K,), jnp.int32),
          pltpu.VMEM((K,), jnp.int32),
          pltpu.VMEM((K,), jnp.int32),
          pltpu.VMEM((K,), _f32),
          pltpu.VMEM((K, C), _f32),
          pltpu.VMEM((K, C), _f32),
          pltpu.VMEM_SHARED((NP, C), _f32),
          pltpu.VMEM_SHARED((NP,), _f32),
          pltpu.SemaphoreType.DMA,
          pltpu.SemaphoreType.DMA,
      ],
      compiler_params=pltpu.CompilerParams(needs_layout_passes=False),
      name=f"sc_gat_edge_c{C}",
  )


def _tc_first(x, W, asrc, adst):
  Cin, C = W.shape

  def body(x_ref, w_ref, asrc_ref, adst_ref, h_ref, asad_ref):
    h = jnp.dot(x_ref[...], w_ref[...], preferred_element_type=_f32,
                precision=lax.Precision.HIGHEST)
    h_ref[...] = h
    a_s = jnp.sum(h * asrc_ref[...], axis=1)
    a_d = jnp.sum(h * adst_ref[...], axis=1)
    asad_ref[...] = jnp.stack([a_s, a_d], axis=1)

  return pl.pallas_call(
      body,
      grid=(NBLK,),
      in_specs=[
          pl.BlockSpec((BM, Cin), lambda i: (i, 0)),
          pl.BlockSpec((Cin, C), lambda i: (0, 0)),
          pl.BlockSpec((1, C), lambda i: (0, 0)),
          pl.BlockSpec((1, C), lambda i: (0, 0)),
      ],
      out_specs=[
          pl.BlockSpec((BM, C), lambda i: (i, 0)),
          pl.BlockSpec((BM, 2), lambda i: (i, 0)),
      ],
      out_shape=[
          jax.ShapeDtypeStruct((NP, C), _f32),
          jax.ShapeDtypeStruct((NP, 2), _f32),
      ],
      name="tc_first",
  )(x, W, asrc.reshape(1, C), adst.reshape(1, C))


def _tc_combine_stage(acc, den, b, W, asrc, adst, *, relu, want_x,
                      want_latsum=False):
  Cin, C = W.shape

  def body(acc_ref, den_ref, b_ref, w_ref, asrc_ref, adst_ref, *outs):
    dsum = jnp.sum(den_ref[...], axis=0)
    x = (acc_ref[0] + acc_ref[1]) / (dsum[:, None] + 1e-16) + b_ref[...]
    if relu:
      x = jnp.maximum(x, 0.0)
    h = jnp.dot(x, w_ref[...], preferred_element_type=_f32,
                precision=lax.Precision.HIGHEST)
    a_s = jnp.sum(h * asrc_ref[...], axis=1)
    a_d = jnp.sum(h * adst_ref[...], axis=1)
    o = 0
    outs[o][...] = h; o += 1
    outs[o][...] = jnp.stack([a_s, a_d], axis=1); o += 1
    if want_x:
      outs[o][...] = x; o += 1
    if want_latsum:
      rid = pl.program_id(0) * BM + lax.broadcasted_iota(jnp.int32, (BM, 1), 0)
      xm = jnp.where(rid < N, x, 0.0)
      bsum = jnp.dot(jnp.ones((8, BM), _f32) / 8.0, xm,
                     preferred_element_type=_f32,
                     precision=lax.Precision.HIGHEST)
      outs[o][...] = bsum; o += 1

  out_specs = [
      pl.BlockSpec((BM, C), lambda i: (i, 0)),
      pl.BlockSpec((BM, 2), lambda i: (i, 0)),
  ]
  out_shape = [
      jax.ShapeDtypeStruct((NP, C), _f32),
      jax.ShapeDtypeStruct((NP, 2), _f32),
  ]
  if want_x:
    out_specs.append(pl.BlockSpec((BM, Cin), lambda i: (i, 0)))
    out_shape.append(jax.ShapeDtypeStruct((NP, Cin), _f32))
  if want_latsum:
    out_specs.append(pl.BlockSpec((8, Cin), lambda i: (i, 0)))
    out_shape.append(jax.ShapeDtypeStruct((NBLK * 8, Cin), _f32))

  return pl.pallas_call(
      body,
      grid=(NBLK,),
      in_specs=[
          pl.BlockSpec((NC, BM, Cin), lambda i: (0, i, 0)),
          pl.BlockSpec((NC, BM), lambda i: (0, i)),
          pl.BlockSpec((1, Cin), lambda i: (0, 0)),
          pl.BlockSpec((Cin, C), lambda i: (0, 0)),
          pl.BlockSpec((1, C), lambda i: (0, 0)),
          pl.BlockSpec((1, C), lambda i: (0, 0)),
      ],
      out_specs=out_specs,
      out_shape=out_shape,
      name="tc_combine_mm",
  )(acc, den, b.reshape(1, Cin), W, asrc.reshape(1, C), adst.reshape(1, C))


def _tc_final_combine(acc, den, b):
  C = acc.shape[-1]

  def body(acc_ref, den_ref, b_ref, o_ref):
    dsum = jnp.sum(den_ref[...], axis=0)
    o_ref[...] = (acc_ref[0] + acc_ref[1]) / (dsum[:, None] + 1e-16) + b_ref[...]

  return pl.pallas_call(
      body,
      grid=(NBLK,),
      in_specs=[
          pl.BlockSpec((NC, BM, C), lambda i: (0, i, 0)),
          pl.BlockSpec((NC, BM), lambda i: (0, i)),
          pl.BlockSpec((1, C), lambda i: (0, 0)),
      ],
      out_specs=pl.BlockSpec((BM, C), lambda i: (i, 0)),
      out_shape=jax.ShapeDtypeStruct((NP, C), _f32),
      name="tc_final_combine",
  )(acc, den, b.reshape(1, C))


def _tc_classifier(latsum, Wc1, bc1, Wc2, bc2):
  def body(ls_ref, w1_ref, b1_ref, w2_ref, b2_ref, o_ref):
    pooled = jnp.sum(ls_ref[...], axis=0, keepdims=True) / float(N)
    z = jnp.maximum(
        jnp.dot(pooled, w1_ref[...], preferred_element_type=_f32,
                precision=lax.Precision.HIGHEST) + b1_ref[...], 0.0)
    o_ref[...] = jnp.dot(z, w2_ref[...], preferred_element_type=_f32,
                         precision=lax.Precision.HIGHEST) + b2_ref[...]

  return pl.pallas_call(
      body,
      out_shape=jax.ShapeDtypeStruct((1, N_CLS), _f32),
      name="tc_classifier",
  )(latsum, Wc1, bc1.reshape(1, -1), Wc2, bc2.reshape(1, -1))


_sc_edge_128 = _make_sc_edge(128)


@jax.jit
def kernel(x, edge_index, W1, a_src1, a_dst1, b1, W2, a_src2, a_dst2, b2,
           W3, a_src3, a_dst3, b3, W4, a_src4, a_dst4, b4, Wc1, bc1, Wc2, bc2):
  W2 = jnp.pad(W2, ((0, 0), (0, 64)))
  a_src2 = jnp.pad(a_src2, (0, 64))
  a_dst2 = jnp.pad(a_dst2, (0, 64))
  b2 = jnp.pad(b2, (0, 64))
  W3 = jnp.pad(W3, ((0, 64), (0, 0)))
  Wc1 = jnp.pad(Wc1, ((0, 64), (0, 0)))
  x_pad = jnp.zeros((NP, D_IN), _f32).at[:N].set(x)
  loops = jnp.arange(N, dtype=jnp.int32)
  src = jnp.full((ETOT,), N, jnp.int32).at[:EAUG].set(
      jnp.concatenate([edge_index[0], loops]))
  dst = jnp.full((ETOT,), N, jnp.int32).at[:EAUG].set(
      jnp.concatenate([edge_index[1], loops]))

  h1, asad1 = _tc_first(x_pad, W1, a_src1, a_dst1)
  acc1, den1 = _sc_edge_128(h1, asad1.reshape(-1), src, dst)
  h2, asad2 = _tc_combine_stage(acc1, den1, b1, W2, a_src2, a_dst2,
                                relu=True, want_x=False)
  acc2, den2 = _sc_edge_128(h2, asad2.reshape(-1), src, dst)
  h3, asad3, latent_full, latsum = _tc_combine_stage(
      acc2, den2, b2, W3, a_src3, a_dst3, relu=False, want_x=True,
      want_latsum=True)
  acc3, den3 = _sc_edge_128(h3, asad3.reshape(-1), src, dst)
  h4, asad4 = _tc_combine_stage(acc3, den3, b3, W4, a_src4, a_dst4,
                                relu=True, want_x=False)
  acc4, den4 = _sc_edge_128(h4, asad4.reshape(-1), src, dst)
  recon_full = _tc_final_combine(acc4, den4, b4)

  noise_pred = _tc_classifier(latsum, Wc1, bc1, Wc2, bc2)

  return recon_full[:N], latent_full[:N, :D_LAT], noise_pred


# speedup vs baseline: 38.5570x; 1.1861x over previous
import functools

import jax
import jax.numpy as jnp
from jax import lax
from jax.experimental import pallas as pl
from jax.experimental.pallas import tpu as pltpu
from jax.experimental.pallas import tpu_sc as plsc

N = 10000
E = 320000
EAUG = E + N
D_IN, D_HID, D_LAT, N_CLS = 128, 128, 64, 10

NP = 10240
BM = 256
NBLK = NP // BM

NC, NS = 2, 16
NW = NC * NS
K = 96
GCH = 6
NCHUNK = 2 * GCH * (-(-EAUG // (NW * K * 2 * GCH)))
TPW = NCHUNK * K
ETOT = NW * TPW
ROWS_PER_TILE = NP // NS

_f32 = jnp.float32


def _make_sc_edge(C):
  mesh = plsc.VectorSubcoreMesh(core_axis_name="c", subcore_axis_name="s")

  def body(h_hbm, asad_hbm, src_hbm, dst_hbm,
           acc_out, den_out,
           asad_v, srcg_v, dstg_v, ex0_v, ex1_v,
           rows0_v, rows1_v, acc_sh, den_sh,
           semg0, semg1, semd0, semd1, semr0, semr1):
    c = lax.axis_index("c")
    s = lax.axis_index("s")
    wid = s * NC + c

    pltpu.sync_copy(asad_hbm, asad_v)

    zeros16 = jnp.zeros((16,), _f32)
    def _zrow(i, carry):
      rows0_v[i // (C // 16), pl.ds((i % (C // 16)) * 16, 16)] = zeros16
      return carry
    lax.fori_loop(0, K * C // 16, _zrow, 0, unroll=8)
    rbase = s * ROWS_PER_TILE
    def _zacc(i, carry):
      pltpu.sync_copy(rows0_v, acc_sh.at[pl.ds(rbase + i * K, K)])
      return carry
    lax.fori_loop(0, ROWS_PER_TILE // K, _zacc, 0)
    rem = ROWS_PER_TILE % K
    if rem:
      pltpu.sync_copy(rows0_v.at[pl.ds(0, rem)],
                      acc_sh.at[pl.ds(rbase + (ROWS_PER_TILE // K) * K, rem)])
    flat0 = rows0_v.at[0]
    assert ROWS_PER_TILE % C == 0
    def _zden(i, carry):
      pltpu.sync_copy(flat0, den_sh.at[pl.ds(rbase + i * C, C)])
      return carry
    lax.fori_loop(0, ROWS_PER_TILE // C, _zden, 0)
    plsc.subcore_barrier()

    grow0 = wid * NCHUNK

    def _stage_group(g):
      pltpu.sync_copy(src_hbm.at[pl.ds(grow0 + g * GCH, GCH)],
                      srcg_v.at[pl.ds((g % 2) * GCH, GCH)])
      pltpu.sync_copy(dst_hbm.at[pl.ds(grow0 + g * GCH, GCH)],
                      dstg_v.at[pl.ds((g % 2) * GCH, GCH)])

    def _grow(j):
      return ((j // GCH) % 2) * GCH + j % GCH

    def _alpha(j, ex_v):
      r = _grow(j)
      def step(i, carry):
        sv = srcg_v[r, 0, pl.ds(i * 16, 16)]
        dv = dstg_v[r, 0, pl.ds(i * 16, 16)]
        ag = plsc.load_gather(asad_v, [sv * 2])
        dg = plsc.load_gather(asad_v, [dv * 2 + 1])
        al = ag + dg
        al = jnp.maximum(al, al * 0.2)
        ex = jnp.exp(al)
        ex_v[pl.ds(i * 16, 16)] = ex
        return carry
      lax.fori_loop(0, K // 16, step, 0)

    def _scale(rows_ref, ex_v):
      def step(i, carry):
        ex16 = ex_v[pl.ds(i * 16, 16)]
        for jj in range(16):
          e = ex16[jj]
          r = i * 16 + jj
          for cc in range(C // 16):
            sl = pl.ds(cc * 16, 16)
            rows_ref[r, sl] = rows_ref[r, sl] * e
        return carry
      lax.fori_loop(0, K // 16, step, 0)

    def _gather(j, rows_ref, semg):
      pltpu.async_copy(h_hbm.at[srcg_v.at[_grow(j), 0]], rows_ref, semg)

    def _process(j, rows_ref, ex_v, semg, semd, semr):
      _alpha(j, ex_v)
      dv_ref = dstg_v.at[_grow(j), 0]
      dden = pltpu.async_copy(ex_v, den_sh.at[dv_ref], semd, add=True)
      pltpu.make_async_copy(h_hbm.at[srcg_v.at[_grow(j), 0]], rows_ref,
                            semg).wait()
      _scale(rows_ref, ex_v)
      drows = pltpu.async_copy(rows_ref, acc_sh.at[dv_ref], semr, add=True)
      return dden, drows

    def _drain(dden, drows):
      dden.wait()
      drows.wait()

    _stage_group(0)
    _stage_group(1)
    _gather(0, rows0_v, semg0)
    _gather(1, rows1_v, semg1)

    NGRP = NCHUNK // GCH

    def _super(t, carry):
      j0 = 2 * t
      j2 = j0 + 2
      @pl.when(jnp.logical_and(j0 % GCH == 0,
                               jnp.logical_and(j0 > 0,
                                               j0 // GCH + 1 < NGRP)))
      def _():
        _stage_group(j0 // GCH + 1)
      d0 = _process(j0, rows0_v, ex0_v, semg0, semd0, semr0)
      d1 = _process(j0 + 1, rows1_v, ex1_v, semg1, semd1, semr1)
      _drain(*d0)
      @pl.when(j2 < NCHUNK)
      def _():
        _gather(j2, rows0_v, semg0)
      _drain(*d1)
      @pl.when(j2 + 1 < NCHUNK)
      def _():
        _gather(j2 + 1, rows1_v, semg1)
      return carry
    lax.fori_loop(0, NCHUNK // 2, _super, 0)

    plsc.subcore_barrier()
    pltpu.sync_copy(acc_sh.at[pl.ds(rbase, ROWS_PER_TILE)],
                    acc_out.at[c, pl.ds(rbase, ROWS_PER_TILE)])
    pltpu.sync_copy(den_sh.at[pl.ds(rbase, ROWS_PER_TILE)],
                    den_out.at[c, pl.ds(rbase, ROWS_PER_TILE)])

  return pl.kernel(
      body,
      out_type=(jax.ShapeDtypeStruct((NC, NP, C), _f32),
                jax.ShapeDtypeStruct((NC, NP), _f32)),
      mesh=mesh,
      scratch_types=[
          pltpu.VMEM((2 * NP,), _f32),
          pltpu.VMEM((2 * GCH, 1, K), jnp.int32),
          pltpu.VMEM((2 * GCH, 1, K), jnp.int32),
          pltpu.VMEM((K,), _f32),
          pltpu.VMEM((K,), _f32),
          pltpu.VMEM((K, C), _f32),
          pltpu.VMEM((K, C), _f32),
          pltpu.VMEM_SHARED((NP, C), _f32),
          pltpu.VMEM_SHARED((NP,), _f32),
          pltpu.SemaphoreType.DMA,
          pltpu.SemaphoreType.DMA,
          pltpu.SemaphoreType.DMA,
          pltpu.SemaphoreType.DMA,
          pltpu.SemaphoreType.DMA,
          pltpu.SemaphoreType.DMA,
      ],
      compiler_params=pltpu.CompilerParams(needs_layout_passes=False),
      name=f"sc_gat_edge_c{C}",
  )


def _tc_first(x, W, asrc, adst):
  Cin, C = W.shape

  def body(x_ref, w_ref, asrc_ref, adst_ref, h_ref, asad_ref):
    h = jnp.dot(x_ref[...], w_ref[...], preferred_element_type=_f32,
                precision=lax.Precision.HIGHEST)
    h_ref[...] = h
    a_s = jnp.sum(h * asrc_ref[...], axis=1)
    a_d = jnp.sum(h * adst_ref[...], axis=1)
    asad_ref[...] = jnp.stack([a_s, a_d], axis=1)

  return pl.pallas_call(
      body,
      grid=(NBLK,),
      in_specs=[
          pl.BlockSpec((BM, Cin), lambda i: (i, 0)),
          pl.BlockSpec((Cin, C), lambda i: (0, 0)),
          pl.BlockSpec((1, C), lambda i: (0, 0)),
          pl.BlockSpec((1, C), lambda i: (0, 0)),
      ],
      out_specs=[
          pl.BlockSpec((BM, C), lambda i: (i, 0)),
          pl.BlockSpec((BM, 2), lambda i: (i, 0)),
      ],
      out_shape=[
          jax.ShapeDtypeStruct((NP, C), _f32),
          jax.ShapeDtypeStruct((NP, 2), _f32),
      ],
      name="tc_first",
  )(x, W, asrc.reshape(1, C), adst.reshape(1, C))


def _tc_combine_stage(acc, den, b, W, asrc, adst, *, relu, want_x,
                      want_latsum=False):
  Cin, C = W.shape

  def body(acc_ref, den_ref, b_ref, w_ref, asrc_ref, adst_ref, *outs):
    dsum = jnp.sum(den_ref[...], axis=0)
    x = (acc_ref[0] + acc_ref[1]) / (dsum[:, None] + 1e-16) + b_ref[...]
    if relu:
      x = jnp.maximum(x, 0.0)
    h = jnp.dot(x, w_ref[...], preferred_element_type=_f32,
                precision=lax.Precision.HIGHEST)
    a_s = jnp.sum(h * asrc_ref[...], axis=1)
    a_d = jnp.sum(h * adst_ref[...], axis=1)
    o = 0
    outs[o][...] = h; o += 1
    outs[o][...] = jnp.stack([a_s, a_d], axis=1); o += 1
    if want_x:
      outs[o][...] = x; o += 1
    if want_latsum:
      rid = pl.program_id(0) * BM + lax.broadcasted_iota(jnp.int32, (BM, 1), 0)
      xm = jnp.where(rid < N, x, 0.0)
      bsum = jnp.dot(jnp.ones((8, BM), _f32) / 8.0, xm,
                     preferred_element_type=_f32,
                     precision=lax.Precision.HIGHEST)
      outs[o][...] = bsum; o += 1

  out_specs = [
      pl.BlockSpec((BM, C), lambda i: (i, 0)),
      pl.BlockSpec((BM, 2), lambda i: (i, 0)),
  ]
  out_shape = [
      jax.ShapeDtypeStruct((NP, C), _f32),
      jax.ShapeDtypeStruct((NP, 2), _f32),
  ]
  if want_x:
    out_specs.append(pl.BlockSpec((BM, Cin), lambda i: (i, 0)))
    out_shape.append(jax.ShapeDtypeStruct((NP, Cin), _f32))
  if want_latsum:
    out_specs.append(pl.BlockSpec((8, Cin), lambda i: (i, 0)))
    out_shape.append(jax.ShapeDtypeStruct((NBLK * 8, Cin), _f32))

  return pl.pallas_call(
      body,
      grid=(NBLK,),
      in_specs=[
          pl.BlockSpec((NC, BM, Cin), lambda i: (0, i, 0)),
          pl.BlockSpec((NC, BM), lambda i: (0, i)),
          pl.BlockSpec((1, Cin), lambda i: (0, 0)),
          pl.BlockSpec((Cin, C), lambda i: (0, 0)),
          pl.BlockSpec((1, C), lambda i: (0, 0)),
          pl.BlockSpec((1, C), lambda i: (0, 0)),
      ],
      out_specs=out_specs,
      out_shape=out_shape,
      name="tc_combine_mm",
  )(acc, den, b.reshape(1, Cin), W, asrc.reshape(1, C), adst.reshape(1, C))


def _tc_final_combine(acc, den, b):
  C = acc.shape[-1]

  def body(acc_ref, den_ref, b_ref, o_ref):
    dsum = jnp.sum(den_ref[...], axis=0)
    o_ref[...] = (acc_ref[0] + acc_ref[1]) / (dsum[:, None] + 1e-16) + b_ref[...]

  return pl.pallas_call(
      body,
      grid=(NBLK,),
      in_specs=[
          pl.BlockSpec((NC, BM, C), lambda i: (0, i, 0)),
          pl.BlockSpec((NC, BM), lambda i: (0, i)),
          pl.BlockSpec((1, C), lambda i: (0, 0)),
      ],
      out_specs=pl.BlockSpec((BM, C), lambda i: (i, 0)),
      out_shape=jax.ShapeDtypeStruct((NP, C), _f32),
      name="tc_final_combine",
  )(acc, den, b.reshape(1, C))


def _tc_classifier(latsum, Wc1, bc1, Wc2, bc2):
  def body(ls_ref, w1_ref, b1_ref, w2_ref, b2_ref, o_ref):
    pooled = jnp.sum(ls_ref[...], axis=0, keepdims=True) / float(N)
    z = jnp.maximum(
        jnp.dot(pooled, w1_ref[...], preferred_element_type=_f32,
                precision=lax.Precision.HIGHEST) + b1_ref[...], 0.0)
    o_ref[...] = jnp.dot(z, w2_ref[...], preferred_element_type=_f32,
                         precision=lax.Precision.HIGHEST) + b2_ref[...]

  return pl.pallas_call(
      body,
      out_shape=jax.ShapeDtypeStruct((1, N_CLS), _f32),
      name="tc_classifier",
  )(latsum, Wc1, bc1.reshape(1, -1), Wc2, bc2.reshape(1, -1))


_sc_edge_128 = _make_sc_edge(128)


@jax.jit
def kernel(x, edge_index, W1, a_src1, a_dst1, b1, W2, a_src2, a_dst2, b2,
           W3, a_src3, a_dst3, b3, W4, a_src4, a_dst4, b4, Wc1, bc1, Wc2, bc2):
  W2 = jnp.pad(W2, ((0, 0), (0, 64)))
  a_src2 = jnp.pad(a_src2, (0, 64))
  a_dst2 = jnp.pad(a_dst2, (0, 64))
  b2 = jnp.pad(b2, (0, 64))
  W3 = jnp.pad(W3, ((0, 64), (0, 0)))
  Wc1 = jnp.pad(Wc1, ((0, 64), (0, 0)))
  x_pad = jnp.zeros((NP, D_IN), _f32).at[:N].set(x)
  loops = jnp.arange(N, dtype=jnp.int32)
  src = jnp.full((ETOT,), N, jnp.int32).at[:EAUG].set(
      jnp.concatenate([edge_index[0], loops])).reshape(NW * NCHUNK, 1, K)
  dst = jnp.full((ETOT,), N, jnp.int32).at[:EAUG].set(
      jnp.concatenate([edge_index[1], loops])).reshape(NW * NCHUNK, 1, K)

  h1, asad1 = _tc_first(x_pad, W1, a_src1, a_dst1)
  acc1, den1 = _sc_edge_128(h1, asad1.reshape(-1), src, dst)
  h2, asad2 = _tc_combine_stage(acc1, den1, b1, W2, a_src2, a_dst2,
                                relu=True, want_x=False)
  acc2, den2 = _sc_edge_128(h2, asad2.reshape(-1), src, dst)
  h3, asad3, latent_full, latsum = _tc_combine_stage(
      acc2, den2, b2, W3, a_src3, a_dst3, relu=False, want_x=True,
      want_latsum=True)
  acc3, den3 = _sc_edge_128(h3, asad3.reshape(-1), src, dst)
  h4, asad4 = _tc_combine_stage(acc3, den3, b3, W4, a_src4, a_dst4,
                                relu=True, want_x=False)
  acc4, den4 = _sc_edge_128(h4, asad4.reshape(-1), src, dst)
  recon_full = _tc_final_combine(acc4, den4, b4)

  noise_pred = _tc_classifier(latsum, Wc1, bc1, Wc2, bc2)

  return recon_full[:N], latent_full[:N, :D_LAT], noise_pred
